# baseline (speedup 1.0000x reference)
import functools
import math

import jax
import jax.numpy as jnp
from jax import lax
from jax.experimental import pallas as pl
from jax.experimental.pallas import tpu as pltpu
from jax.experimental.pallas import tpu_sc as plsc

N = 10000
DIN = 128
DH = 128
DOUT = 64
E = 320000
NC = 2
NS = 16
NW = NC * NS
EPT = E // NW
K = 80
NCHUNK = EPT // K
KC = 64
NFULL = EPT // KC
NPAIR = NFULL // 2
KT = EPT - NFULL * KC
RPS = N // NS
RPA = 624
RTAIL = N - NS * RPA
ROWB = 400
GRID = N // ROWB
BN_SCALE = 1.0 / math.sqrt(1.0 + 1e-5)

_MESH = plsc.VectorSubcoreMesh(core_axis_name="core", subcore_axis_name="subcore")


@functools.partial(
    pl.kernel,
    out_type=jax.ShapeDtypeStruct((NC, N, DH), jnp.float32),
    mesh=_MESH,
    scratch_types=[
        pltpu.VMEM((KC,), jnp.int32),
        pltpu.VMEM((KC,), jnp.int32),
        pltpu.VMEM((KT,), jnp.int32),
        pltpu.VMEM((KT,), jnp.int32),
        pltpu.VMEM((KC, DH), jnp.float32),
        pltpu.VMEM((KT, DH), jnp.float32),
        pltpu.VMEM_SHARED((N, DH), jnp.float32),
        pltpu.SemaphoreType.DMA,
        pltpu.SemaphoreType.DMA,
    ],
)
def _agg_kernel(g_hbm, src_hbm, dst_hbm, part_hbm, didx0, didx1,
                sidxt, didxt, rows0, rowst, acc, sg0, sg1):
    c = lax.axis_index("core")
    s = lax.axis_index("subcore")
    wid = c * NS + s
    r0 = s * RPA
    base0 = wid * EPT

    pltpu.sync_copy(g_hbm.at[pl.ds(r0, RPA)], acc.at[pl.ds(r0, RPA)])

    @pl.when(s == NS - 1)
    def _():
        pltpu.sync_copy(g_hbm.at[pl.ds(NS * RPA, RTAIL)],
                        acc.at[pl.ds(NS * RPA, RTAIL)])

    plsc.subcore_barrier()

    @pl.loop(0, NFULL)
    def _(i):
        b = base0 + i * KC
        pltpu.sync_copy(src_hbm.at[pl.ds(b, KC)], didx1)
        pltpu.sync_copy(dst_hbm.at[pl.ds(b, KC)], didx0)
        pltpu.async_copy(g_hbm.at[didx1], rows0, sg0).wait()
        pltpu.sync_copy(rows0, acc.at[didx0], add=True)

    pltpu.sync_copy(src_hbm.at[pl.ds(base0 + NFULL * KC, KT)], sidxt)
    pltpu.sync_copy(dst_hbm.at[pl.ds(base0 + NFULL * KC, KT)], didxt)
    pltpu.async_copy(g_hbm.at[sidxt], rowst, sg1).wait()
    pltpu.sync_copy(rowst, acc.at[didxt], add=True)

    plsc.subcore_barrier()
    pltpu.sync_copy(acc.at[pl.ds(r0, RPA)], part_hbm.at[c, pl.ds(r0, RPA)])

    @pl.when(s == NS - 1)
    def _():
        pltpu.sync_copy(acc.at[pl.ds(NS * RPA, RTAIL)],
                        part_hbm.at[c, pl.ds(NS * RPA, RTAIL)])


def _dis_from_deg(deg_blk):
    d = deg_blk[0][:, 0:1] + deg_blk[1][:, 0:1] - 1.0
    return 1.0 / jnp.sqrt(d)


def _mm(a, b):
    return lax.dot_general(a, b, (((1,), (0,)), ((), ())),
                           precision=lax.Precision.HIGHEST,
                           preferred_element_type=jnp.float32)


def _mm1_body(x_ref, w_ref, h_ref):
    h_ref[...] = _mm(x_ref[...], w_ref[...])


def _scale_body(h_ref, deg_ref, g_ref):
    g_ref[...] = h_ref[...] * _dis_from_deg(deg_ref[...])


def _mid_body(part_ref, g_ref, deg_ref, b_ref, gam_ref, bet_ref, w_ref, o_ref):
    dis = _dis_from_deg(deg_ref[...])
    p = part_ref[...]
    t = (p[0] + p[1] - g_ref[...]) * dis + b_ref[...]
    t = t * (BN_SCALE * gam_ref[...]) + bet_ref[...]
    e = jnp.where(t > 0, t, jnp.exp(t) - 1.0)
    o_ref[...] = _mm(e, w_ref[...]) * dis


def _out_body(part_ref, g_ref, deg_ref, b_ref, gam_ref, bet_ref, w_ref, bp_ref, o_ref):
    dis = _dis_from_deg(deg_ref[...])
    p = part_ref[...]
    t = (p[0] + p[1] - g_ref[...]) * dis + b_ref[...]
    t = t * (BN_SCALE * gam_ref[...]) + bet_ref[...]
    e = jnp.where(t > 0, t, jnp.exp(t) - 1.0)
    o_ref[...] = _mm(e, w_ref[...]) + bp_ref[...]


def _rows_spec(d):
    return pl.BlockSpec((ROWB, d), lambda i: (i, 0))


def _full_spec(shape):
    nd = len(shape)
    return pl.BlockSpec(shape, lambda i, _nd=nd: (0,) * _nd)


def _part_spec(d):
    return pl.BlockSpec((NC, ROWB, d), lambda i: (0, i, 0))


_DEG_SPEC = pl.BlockSpec((NC, ROWB, DH), lambda i: (0, i, 0))


def _tc_call(body, in_specs, out_d):
    return pl.pallas_call(
        body,
        grid=(GRID,),
        in_specs=in_specs,
        out_specs=_rows_spec(out_d),
        out_shape=jax.ShapeDtypeStruct((N, out_d), jnp.float32),
    )


def kernel(x, edge_index, W1, b1, g1, be1, W2, b2, g2, be2, Wp, bp):
    src = edge_index[0].astype(jnp.int32)
    dst = edge_index[1].astype(jnp.int32)
    b1r, gm1, bt1 = b1.reshape(1, DH), g1.reshape(1, DH), be1.reshape(1, DH)
    b2r, gm2, bt2 = b2.reshape(1, DH), g2.reshape(1, DH), be2.reshape(1, DH)
    bpr = bp.reshape(1, DOUT)

    ones_n = jnp.ones((N, DH), jnp.float32)
    deg_part = _agg_kernel(ones_n, dst, dst)

    h1 = _tc_call(_mm1_body, [_rows_spec(DIN), _full_spec((DIN, DH))], DH)(x, W1)
    g1s = _tc_call(_scale_body, [_rows_spec(DH), _DEG_SPEC], DH)(h1, deg_part)

    part1 = _agg_kernel(g1s, src, dst)

    g2s = _tc_call(
        _mid_body,
        [_part_spec(DH), _rows_spec(DH), _DEG_SPEC, _full_spec((1, DH)),
         _full_spec((1, DH)), _full_spec((1, DH)), _full_spec((DH, DH))],
        DH,
    )(part1, g1s, deg_part, b1r, gm1, bt1, W2)

    part2 = _agg_kernel(g2s, src, dst)

    out = _tc_call(
        _out_body,
        [_part_spec(DH), _rows_spec(DH), _DEG_SPEC, _full_spec((1, DH)),
         _full_spec((1, DH)), _full_spec((1, DH)), _full_spec((DH, DOUT)),
         _full_spec((1, DOUT))],
        DOUT,
    )(part2, g2s, deg_part, b2r, gm2, bt2, Wp, bpr)

    return out


# speedup vs baseline: 1.8700x; 1.8700x over previous
import functools
import math

import jax
import jax.numpy as jnp
from jax import lax
from jax.experimental import pallas as pl
from jax.experimental.pallas import tpu as pltpu
from jax.experimental.pallas import tpu_sc as plsc

N = 10000
DIN = 128
DH = 128
DOUT = 64
E = 320000
NC = 2
NS = 16
NW = NC * NS
EPT = E // NW
K = 80
NCHUNK = EPT // K
KC = 64
NFULL = EPT // KC
NPAIR = NFULL // 2
KT = EPT - NFULL * KC
RPS = N // NS
RPA = 624
RTAIL = N - NS * RPA
ROWB = 400
GRID = N // ROWB
BN_SCALE = 1.0 / math.sqrt(1.0 + 1e-5)

_MESH = plsc.VectorSubcoreMesh(core_axis_name="core", subcore_axis_name="subcore")


@functools.partial(
    pl.kernel,
    out_type=jax.ShapeDtypeStruct((NC, N, DH), jnp.float32),
    mesh=_MESH,
    scratch_types=[
        pltpu.VMEM((EPT,), jnp.int32),
        pltpu.VMEM((KC,), jnp.int32),
        pltpu.VMEM((KC,), jnp.int32),
        pltpu.VMEM((KT,), jnp.int32),
        pltpu.VMEM((KC, DH), jnp.float32),
        pltpu.VMEM((KC, DH), jnp.float32),
        pltpu.VMEM((KT, DH), jnp.float32),
        pltpu.VMEM_SHARED((N, DH), jnp.float32),
        pltpu.SemaphoreType.DMA,
        pltpu.SemaphoreType.DMA,
        pltpu.SemaphoreType.DMA,
        pltpu.SemaphoreType.DMA,
    ],
)
def _agg_kernel(g_hbm, src_hbm, dst_hbm, part_hbm, sidx_all, didx0, didx1,
                didxt, rows0, rows1, rowst, acc, sd0, sd1, sg0, sg1):
    c = lax.axis_index("core")
    s = lax.axis_index("subcore")
    wid = c * NS + s
    r0 = s * RPA
    base0 = wid * EPT

    pltpu.sync_copy(g_hbm.at[pl.ds(r0, RPA)], acc.at[pl.ds(r0, RPA)])

    @pl.when(s == NS - 1)
    def _():
        pltpu.sync_copy(g_hbm.at[pl.ds(NS * RPA, RTAIL)],
                        acc.at[pl.ds(NS * RPA, RTAIL)])

    pltpu.sync_copy(src_hbm.at[pl.ds(base0, EPT)], sidx_all)
    plsc.subcore_barrier()

    def _didx(ck, buf, sem):
        return pltpu.make_async_copy(
            dst_hbm.at[pl.ds(base0 + ck * KC, KC)], buf, sem)

    def _gath(ck, buf, sem):
        return pltpu.make_async_copy(
            g_hbm.at[sidx_all.at[pl.ds(ck * KC, KC)]], buf, sem)

    _didx(0, didx0, sd0).start()
    _gath(0, rows0, sg0).start()

    @pl.loop(0, NPAIR)
    def _(t):
        a = 2 * t
        _didx(a, didx0, sd0).wait()
        _gath(a, rows0, sg0).wait()
        _didx(a + 1, didx1, sd1).start()
        _gath(a + 1, rows1, sg1).start()
        pltpu.sync_copy(rows0, acc.at[didx0], add=True)
        _didx(a + 1, didx1, sd1).wait()
        _gath(a + 1, rows1, sg1).wait()

        @pl.when(t < NPAIR - 1)
        def _():
            _didx(a + 2, didx0, sd0).start()
            _gath(a + 2, rows0, sg0).start()

        pltpu.sync_copy(rows1, acc.at[didx1], add=True)

    pltpu.sync_copy(dst_hbm.at[pl.ds(base0 + NFULL * KC, KT)], didxt)
    pltpu.sync_copy(g_hbm.at[sidx_all.at[pl.ds(NFULL * KC, KT)]], rowst)
    pltpu.sync_copy(rowst, acc.at[didxt], add=True)

    plsc.subcore_barrier()
    pltpu.sync_copy(acc.at[pl.ds(r0, RPA)], part_hbm.at[c, pl.ds(r0, RPA)])

    @pl.when(s == NS - 1)
    def _():
        pltpu.sync_copy(acc.at[pl.ds(NS * RPA, RTAIL)],
                        part_hbm.at[c, pl.ds(NS * RPA, RTAIL)])


def _dis_from_deg(deg_blk):
    d = deg_blk[0][:, 0:1] + deg_blk[1][:, 0:1] - 1.0
    return 1.0 / jnp.sqrt(d)


def _mm(a, b):
    return lax.dot_general(a, b, (((1,), (0,)), ((), ())),
                           precision=lax.Precision.HIGHEST,
                           preferred_element_type=jnp.float32)


def _mm1_body(x_ref, w_ref, h_ref):
    h_ref[...] = _mm(x_ref[...], w_ref[...])


def _scale_body(h_ref, deg_ref, g_ref):
    g_ref[...] = h_ref[...] * _dis_from_deg(deg_ref[...])


def _mid_body(part_ref, g_ref, deg_ref, b_ref, gam_ref, bet_ref, w_ref, o_ref):
    dis = _dis_from_deg(deg_ref[...])
    p = part_ref[...]
    t = (p[0] + p[1] - g_ref[...]) * dis + b_ref[...]
    t = t * (BN_SCALE * gam_ref[...]) + bet_ref[...]
    e = jnp.where(t > 0, t, jnp.exp(t) - 1.0)
    o_ref[...] = _mm(e, w_ref[...]) * dis


def _out_body(part_ref, g_ref, deg_ref, b_ref, gam_ref, bet_ref, w_ref, bp_ref, o_ref):
    dis = _dis_from_deg(deg_ref[...])
    p = part_ref[...]
    t = (p[0] + p[1] - g_ref[...]) * dis + b_ref[...]
    t = t * (BN_SCALE * gam_ref[...]) + bet_ref[...]
    e = jnp.where(t > 0, t, jnp.exp(t) - 1.0)
    o_ref[...] = _mm(e, w_ref[...]) + bp_ref[...]


def _rows_spec(d):
    return pl.BlockSpec((ROWB, d), lambda i: (i, 0))


def _full_spec(shape):
    nd = len(shape)
    return pl.BlockSpec(shape, lambda i, _nd=nd: (0,) * _nd)


def _part_spec(d):
    return pl.BlockSpec((NC, ROWB, d), lambda i: (0, i, 0))


_DEG_SPEC = pl.BlockSpec((NC, ROWB, DH), lambda i: (0, i, 0))


def _tc_call(body, in_specs, out_d):
    return pl.pallas_call(
        body,
        grid=(GRID,),
        in_specs=in_specs,
        out_specs=_rows_spec(out_d),
        out_shape=jax.ShapeDtypeStruct((N, out_d), jnp.float32),
    )


def kernel(x, edge_index, W1, b1, g1, be1, W2, b2, g2, be2, Wp, bp):
    src = edge_index[0].astype(jnp.int32)
    dst = edge_index[1].astype(jnp.int32)
    b1r, gm1, bt1 = b1.reshape(1, DH), g1.reshape(1, DH), be1.reshape(1, DH)
    b2r, gm2, bt2 = b2.reshape(1, DH), g2.reshape(1, DH), be2.reshape(1, DH)
    bpr = bp.reshape(1, DOUT)

    ones_n = jnp.ones((N, DH), jnp.float32)
    deg_part = _agg_kernel(ones_n, dst, dst)

    h1 = _tc_call(_mm1_body, [_rows_spec(DIN), _full_spec((DIN, DH))], DH)(x, W1)
    g1s = _tc_call(_scale_body, [_rows_spec(DH), _DEG_SPEC], DH)(h1, deg_part)

    part1 = _agg_kernel(g1s, src, dst)

    g2s = _tc_call(
        _mid_body,
        [_part_spec(DH), _rows_spec(DH), _DEG_SPEC, _full_spec((1, DH)),
         _full_spec((1, DH)), _full_spec((1, DH)), _full_spec((DH, DH))],
        DH,
    )(part1, g1s, deg_part, b1r, gm1, bt1, W2)

    part2 = _agg_kernel(g2s, src, dst)

    out = _tc_call(
        _out_body,
        [_part_spec(DH), _rows_spec(DH), _DEG_SPEC, _full_spec((1, DH)),
         _full_spec((1, DH)), _full_spec((1, DH)), _full_spec((DH, DOUT)),
         _full_spec((1, DOUT))],
        DOUT,
    )(part2, g2s, deg_part, b2r, gm2, bt2, Wp, bpr)

    return out


# speedup vs baseline: 2.3794x; 1.2724x over previous
import functools
import math

import jax
import jax.numpy as jnp
from jax import lax
from jax.experimental import pallas as pl
from jax.experimental.pallas import tpu as pltpu
from jax.experimental.pallas import tpu_sc as plsc

N = 10000
DIN = 128
DH = 128
DOUT = 64
E = 320000
NC = 2
NS = 16
NW = NC * NS
EPT = E // NW
K = 80
NCHUNK = EPT // K
KC = 128
NFULL = EPT // KC
NPAIR = NFULL // 2
KT = EPT - NFULL * KC
RPS = N // NS
RPA = 624
RTAIL = N - NS * RPA
ROWB = 400
GRID = N // ROWB
BN_SCALE = 1.0 / math.sqrt(1.0 + 1e-5)

_MESH = plsc.VectorSubcoreMesh(core_axis_name="core", subcore_axis_name="subcore")


@functools.partial(
    pl.kernel,
    out_type=jax.ShapeDtypeStruct((NC, N, DH), jnp.float32),
    mesh=_MESH,
    scratch_types=[
        pltpu.VMEM((EPT,), jnp.int32),
        pltpu.VMEM((KC,), jnp.int32),
        pltpu.VMEM((KC,), jnp.int32),
        pltpu.VMEM((KT,), jnp.int32),
        pltpu.VMEM((KC, DH), jnp.float32),
        pltpu.VMEM((KC, DH), jnp.float32),
        pltpu.VMEM((KT, DH), jnp.float32),
        pltpu.VMEM_SHARED((N, DH), jnp.float32),
        pltpu.SemaphoreType.DMA,
        pltpu.SemaphoreType.DMA,
        pltpu.SemaphoreType.DMA,
        pltpu.SemaphoreType.DMA,
    ],
)
def _agg_kernel(g_hbm, src_hbm, dst_hbm, part_hbm, sidx_all, didx0, didx1,
                didxt, rows0, rows1, rowst, acc, sd0, sd1, sg0, sg1):
    c = lax.axis_index("core")
    s = lax.axis_index("subcore")
    wid = c * NS + s
    r0 = s * RPA
    base0 = wid * EPT

    pltpu.sync_copy(g_hbm.at[pl.ds(r0, RPA)], acc.at[pl.ds(r0, RPA)])

    @pl.when(s == NS - 1)
    def _():
        pltpu.sync_copy(g_hbm.at[pl.ds(NS * RPA, RTAIL)],
                        acc.at[pl.ds(NS * RPA, RTAIL)])

    pltpu.sync_copy(src_hbm.at[pl.ds(base0, EPT)], sidx_all)
    plsc.subcore_barrier()

    def _didx(ck, buf, sem):
        return pltpu.make_async_copy(
            dst_hbm.at[pl.ds(base0 + ck * KC, KC)], buf, sem)

    def _gath(ck, buf, sem):
        return pltpu.make_async_copy(
            g_hbm.at[sidx_all.at[pl.ds(ck * KC, KC)]], buf, sem)

    _didx(0, didx0, sd0).start()
    _gath(0, rows0, sg0).start()

    @pl.loop(0, NPAIR)
    def _(t):
        a = 2 * t
        _didx(a, didx0, sd0).wait()
        _gath(a, rows0, sg0).wait()
        _didx(a + 1, didx1, sd1).start()
        _gath(a + 1, rows1, sg1).start()
        pltpu.sync_copy(rows0, acc.at[didx0], add=True)
        _didx(a + 1, didx1, sd1).wait()
        _gath(a + 1, rows1, sg1).wait()

        @pl.when(t < NPAIR - 1)
        def _():
            _didx(a + 2, didx0, sd0).start()
            _gath(a + 2, rows0, sg0).start()

        pltpu.sync_copy(rows1, acc.at[didx1], add=True)

    pltpu.sync_copy(dst_hbm.at[pl.ds(base0 + NFULL * KC, KT)], didxt)
    pltpu.sync_copy(g_hbm.at[sidx_all.at[pl.ds(NFULL * KC, KT)]], rowst)
    pltpu.sync_copy(rowst, acc.at[didxt], add=True)

    plsc.subcore_barrier()
    pltpu.sync_copy(acc.at[pl.ds(r0, RPA)], part_hbm.at[c, pl.ds(r0, RPA)])

    @pl.when(s == NS - 1)
    def _():
        pltpu.sync_copy(acc.at[pl.ds(NS * RPA, RTAIL)],
                        part_hbm.at[c, pl.ds(NS * RPA, RTAIL)])


def _dis_from_deg(deg_blk):
    d = deg_blk[0][:, 0:1] + deg_blk[1][:, 0:1] - 1.0
    return 1.0 / jnp.sqrt(d)


def _mm(a, b):
    return lax.dot_general(a, b, (((1,), (0,)), ((), ())),
                           precision=lax.Precision.HIGHEST,
                           preferred_element_type=jnp.float32)


def _mm1_body(x_ref, w_ref, h_ref):
    h_ref[...] = _mm(x_ref[...], w_ref[...])


def _scale_body(h_ref, deg_ref, g_ref):
    g_ref[...] = h_ref[...] * _dis_from_deg(deg_ref[...])


def _mid_body(part_ref, g_ref, deg_ref, b_ref, gam_ref, bet_ref, w_ref, o_ref):
    dis = _dis_from_deg(deg_ref[...])
    p = part_ref[...]
    t = (p[0] + p[1] - g_ref[...]) * dis + b_ref[...]
    t = t * (BN_SCALE * gam_ref[...]) + bet_ref[...]
    e = jnp.where(t > 0, t, jnp.exp(t) - 1.0)
    o_ref[...] = _mm(e, w_ref[...]) * dis


def _out_body(part_ref, g_ref, deg_ref, b_ref, gam_ref, bet_ref, w_ref, bp_ref, o_ref):
    dis = _dis_from_deg(deg_ref[...])
    p = part_ref[...]
    t = (p[0] + p[1] - g_ref[...]) * dis + b_ref[...]
    t = t * (BN_SCALE * gam_ref[...]) + bet_ref[...]
    e = jnp.where(t > 0, t, jnp.exp(t) - 1.0)
    o_ref[...] = _mm(e, w_ref[...]) + bp_ref[...]


def _rows_spec(d):
    return pl.BlockSpec((ROWB, d), lambda i: (i, 0))


def _full_spec(shape):
    nd = len(shape)
    return pl.BlockSpec(shape, lambda i, _nd=nd: (0,) * _nd)


def _part_spec(d):
    return pl.BlockSpec((NC, ROWB, d), lambda i: (0, i, 0))


_DEG_SPEC = pl.BlockSpec((NC, ROWB, DH), lambda i: (0, i, 0))


def _tc_call(body, in_specs, out_d):
    return pl.pallas_call(
        body,
        grid=(GRID,),
        in_specs=in_specs,
        out_specs=_rows_spec(out_d),
        out_shape=jax.ShapeDtypeStruct((N, out_d), jnp.float32),
    )


def kernel(x, edge_index, W1, b1, g1, be1, W2, b2, g2, be2, Wp, bp):
    src = edge_index[0].astype(jnp.int32)
    dst = edge_index[1].astype(jnp.int32)
    b1r, gm1, bt1 = b1.reshape(1, DH), g1.reshape(1, DH), be1.reshape(1, DH)
    b2r, gm2, bt2 = b2.reshape(1, DH), g2.reshape(1, DH), be2.reshape(1, DH)
    bpr = bp.reshape(1, DOUT)

    ones_n = jnp.ones((N, DH), jnp.float32)
    deg_part = _agg_kernel(ones_n, dst, dst)

    h1 = _tc_call(_mm1_body, [_rows_spec(DIN), _full_spec((DIN, DH))], DH)(x, W1)
    g1s = _tc_call(_scale_body, [_rows_spec(DH), _DEG_SPEC], DH)(h1, deg_part)

    part1 = _agg_kernel(g1s, src, dst)

    g2s = _tc_call(
        _mid_body,
        [_part_spec(DH), _rows_spec(DH), _DEG_SPEC, _full_spec((1, DH)),
         _full_spec((1, DH)), _full_spec((1, DH)), _full_spec((DH, DH))],
        DH,
    )(part1, g1s, deg_part, b1r, gm1, bt1, W2)

    part2 = _agg_kernel(g2s, src, dst)

    out = _tc_call(
        _out_body,
        [_part_spec(DH), _rows_spec(DH), _DEG_SPEC, _full_spec((1, DH)),
         _full_spec((1, DH)), _full_spec((1, DH)), _full_spec((DH, DOUT)),
         _full_spec((1, DOUT))],
        DOUT,
    )(part2, g2s, deg_part, b2r, gm2, bt2, Wp, bpr)

    return out


# speedup vs baseline: 2.6929x; 1.1318x over previous
import functools
import math

import jax
import jax.numpy as jnp
from jax import lax
from jax.experimental import pallas as pl
from jax.experimental.pallas import tpu as pltpu
from jax.experimental.pallas import tpu_sc as plsc

N = 10000
DIN = 128
DH = 128
DOUT = 64
E = 320000
NC = 2
NS = 16
NW = NC * NS
EPT = E // NW
K = 80
NCHUNK = EPT // K
KC = 128
NFULL = EPT // KC
NPAIR = NFULL // 2
KT = EPT - NFULL * KC
RPS = N // NS
RPA = 624
RTAIL = N - NS * RPA
ROWB = 400
GRID = N // ROWB
BN_SCALE = 1.0 / math.sqrt(1.0 + 1e-5)

_MESH = plsc.VectorSubcoreMesh(core_axis_name="core", subcore_axis_name="subcore")


@functools.partial(
    pl.kernel,
    out_type=jax.ShapeDtypeStruct((NC, N, DH), jnp.float32),
    mesh=_MESH,
    scratch_types=[
        pltpu.VMEM((EPT,), jnp.int32),
        pltpu.VMEM((KC,), jnp.int32),
        pltpu.VMEM((KC,), jnp.int32),
        pltpu.VMEM((KT,), jnp.int32),
        pltpu.VMEM((KC, DH), jnp.float32),
        pltpu.VMEM((KC, DH), jnp.float32),
        pltpu.VMEM((KT, DH), jnp.float32),
        pltpu.VMEM_SHARED((N, DH), jnp.float32),
        pltpu.SemaphoreType.DMA,
        pltpu.SemaphoreType.DMA,
        pltpu.SemaphoreType.DMA,
        pltpu.SemaphoreType.DMA,
    ],
)
def _agg_kernel(g_hbm, src_hbm, dst_hbm, part_hbm, sidx_all, didx0, didx1,
                didxt, rows0, rows1, rowst, acc, sd0, sd1, sg0, sg1):
    c = lax.axis_index("core")
    s = lax.axis_index("subcore")
    wid = c * NS + s
    r0 = s * RPA
    base0 = wid * EPT

    pltpu.sync_copy(g_hbm.at[pl.ds(r0, RPA)], acc.at[pl.ds(r0, RPA)])

    @pl.when(s == NS - 1)
    def _():
        pltpu.sync_copy(g_hbm.at[pl.ds(NS * RPA, RTAIL)],
                        acc.at[pl.ds(NS * RPA, RTAIL)])

    pltpu.sync_copy(src_hbm.at[pl.ds(base0, EPT)], sidx_all)
    plsc.subcore_barrier()

    def _didx(ck, buf, sem):
        return pltpu.make_async_copy(
            dst_hbm.at[pl.ds(base0 + ck * KC, KC)], buf, sem)

    def _gath(ck, buf, sem):
        return pltpu.make_async_copy(
            g_hbm.at[sidx_all.at[pl.ds(ck * KC, KC)]], buf, sem)

    _didx(0, didx0, sd0).start()
    _gath(0, rows0, sg0).start()

    @pl.loop(0, NPAIR)
    def _(t):
        a = 2 * t
        _didx(a, didx0, sd0).wait()
        _gath(a, rows0, sg0).wait()
        _didx(a + 1, didx1, sd1).start()
        _gath(a + 1, rows1, sg1).start()
        pltpu.sync_copy(rows0, acc.at[didx0], add=True)
        _didx(a + 1, didx1, sd1).wait()
        _gath(a + 1, rows1, sg1).wait()

        @pl.when(t < NPAIR - 1)
        def _():
            _didx(a + 2, didx0, sd0).start()
            _gath(a + 2, rows0, sg0).start()

        pltpu.sync_copy(rows1, acc.at[didx1], add=True)

    pltpu.sync_copy(dst_hbm.at[pl.ds(base0 + NFULL * KC, KT)], didxt)
    pltpu.sync_copy(g_hbm.at[sidx_all.at[pl.ds(NFULL * KC, KT)]], rowst)
    pltpu.sync_copy(rowst, acc.at[didxt], add=True)

    plsc.subcore_barrier()
    pltpu.sync_copy(acc.at[pl.ds(r0, RPA)], part_hbm.at[c, pl.ds(r0, RPA)])

    @pl.when(s == NS - 1)
    def _():
        pltpu.sync_copy(acc.at[pl.ds(NS * RPA, RTAIL)],
                        part_hbm.at[c, pl.ds(NS * RPA, RTAIL)])


@functools.partial(
    pl.kernel,
    out_type=jax.ShapeDtypeStruct((NC, N, DH), jnp.float32),
    mesh=_MESH,
    scratch_types=[
        pltpu.VMEM((KC,), jnp.int32),
        pltpu.VMEM((KC,), jnp.int32),
        pltpu.VMEM((KT,), jnp.int32),
        pltpu.VMEM((KC, DH), jnp.float32),
        pltpu.VMEM_SHARED((N, DH), jnp.float32),
        pltpu.SemaphoreType.DMA,
        pltpu.SemaphoreType.DMA,
    ],
)
def _deg_kernel(ones_hbm, dst_hbm, part_hbm, didx0, didx1, didxt, ones,
                acc, sd0, sd1):
    c = lax.axis_index("core")
    s = lax.axis_index("subcore")
    wid = c * NS + s
    r0 = s * RPA
    base0 = wid * EPT

    pltpu.sync_copy(ones_hbm.at[pl.ds(r0, RPA)], acc.at[pl.ds(r0, RPA)])

    @pl.when(s == NS - 1)
    def _():
        pltpu.sync_copy(ones_hbm.at[pl.ds(NS * RPA, RTAIL)],
                        acc.at[pl.ds(NS * RPA, RTAIL)])

    pltpu.sync_copy(ones_hbm.at[pl.ds(0, KC)], ones)
    plsc.subcore_barrier()

    def _didx(ck, buf, sem):
        return pltpu.make_async_copy(
            dst_hbm.at[pl.ds(base0 + ck * KC, KC)], buf, sem)

    _didx(0, didx0, sd0).start()

    @pl.loop(0, NPAIR)
    def _(t):
        a = 2 * t
        _didx(a, didx0, sd0).wait()
        _didx(a + 1, didx1, sd1).start()
        pltpu.sync_copy(ones, acc.at[didx0], add=True)
        _didx(a + 1, didx1, sd1).wait()

        @pl.when(t < NPAIR - 1)
        def _():
            _didx(a + 2, didx0, sd0).start()

        pltpu.sync_copy(ones, acc.at[didx1], add=True)

    pltpu.sync_copy(dst_hbm.at[pl.ds(base0 + NFULL * KC, KT)], didxt)
    pltpu.sync_copy(ones.at[pl.ds(0, KT)], acc.at[didxt], add=True)

    plsc.subcore_barrier()
    pltpu.sync_copy(acc.at[pl.ds(r0, RPA)], part_hbm.at[c, pl.ds(r0, RPA)])

    @pl.when(s == NS - 1)
    def _():
        pltpu.sync_copy(acc.at[pl.ds(NS * RPA, RTAIL)],
                        part_hbm.at[c, pl.ds(NS * RPA, RTAIL)])


def _dis_from_deg(deg_blk):
    d = deg_blk[0][:, 0:1] + deg_blk[1][:, 0:1] - 1.0
    return 1.0 / jnp.sqrt(d)


def _mm(a, b):
    return lax.dot_general(a, b, (((1,), (0,)), ((), ())),
                           precision=lax.Precision.HIGHEST,
                           preferred_element_type=jnp.float32)


def _mm1_body(x_ref, w_ref, deg_ref, g_ref):
    g_ref[...] = _mm(x_ref[...], w_ref[...]) * _dis_from_deg(deg_ref[...])


def _mid_body(part_ref, g_ref, deg_ref, b_ref, gam_ref, bet_ref, w_ref, o_ref):
    dis = _dis_from_deg(deg_ref[...])
    p = part_ref[...]
    t = (p[0] + p[1] - g_ref[...]) * dis + b_ref[...]
    t = t * (BN_SCALE * gam_ref[...]) + bet_ref[...]
    e = jnp.where(t > 0, t, jnp.exp(t) - 1.0)
    o_ref[...] = _mm(e, w_ref[...]) * dis


def _out_body(part_ref, g_ref, deg_ref, b_ref, gam_ref, bet_ref, w_ref, bp_ref, o_ref):
    dis = _dis_from_deg(deg_ref[...])
    p = part_ref[...]
    t = (p[0] + p[1] - g_ref[...]) * dis + b_ref[...]
    t = t * (BN_SCALE * gam_ref[...]) + bet_ref[...]
    e = jnp.where(t > 0, t, jnp.exp(t) - 1.0)
    o_ref[...] = _mm(e, w_ref[...]) + bp_ref[...]


def _rows_spec(d):
    return pl.BlockSpec((ROWB, d), lambda i: (i, 0))


def _full_spec(shape):
    nd = len(shape)
    return pl.BlockSpec(shape, lambda i, _nd=nd: (0,) * _nd)


def _part_spec(d):
    return pl.BlockSpec((NC, ROWB, d), lambda i: (0, i, 0))


_DEG_SPEC = pl.BlockSpec((NC, ROWB, DH), lambda i: (0, i, 0))


def _tc_call(body, in_specs, out_d):
    return pl.pallas_call(
        body,
        grid=(GRID,),
        in_specs=in_specs,
        out_specs=_rows_spec(out_d),
        out_shape=jax.ShapeDtypeStruct((N, out_d), jnp.float32),
    )


def kernel(x, edge_index, W1, b1, g1, be1, W2, b2, g2, be2, Wp, bp):
    src = edge_index[0].astype(jnp.int32)
    dst = edge_index[1].astype(jnp.int32)
    b1r, gm1, bt1 = b1.reshape(1, DH), g1.reshape(1, DH), be1.reshape(1, DH)
    b2r, gm2, bt2 = b2.reshape(1, DH), g2.reshape(1, DH), be2.reshape(1, DH)
    bpr = bp.reshape(1, DOUT)

    ones_n = jnp.ones((N, DH), jnp.float32)
    deg_part = _deg_kernel(ones_n, dst)

    g1s = _tc_call(_mm1_body,
                   [_rows_spec(DIN), _full_spec((DIN, DH)), _DEG_SPEC],
                   DH)(x, W1, deg_part)

    part1 = _agg_kernel(g1s, src, dst)

    g2s = _tc_call(
        _mid_body,
        [_part_spec(DH), _rows_spec(DH), _DEG_SPEC, _full_spec((1, DH)),
         _full_spec((1, DH)), _full_spec((1, DH)), _full_spec((DH, DH))],
        DH,
    )(part1, g1s, deg_part, b1r, gm1, bt1, W2)

    part2 = _agg_kernel(g2s, src, dst)

    out = _tc_call(
        _out_body,
        [_part_spec(DH), _rows_spec(DH), _DEG_SPEC, _full_spec((1, DH)),
         _full_spec((1, DH)), _full_spec((1, DH)), _full_spec((DH, DOUT)),
         _full_spec((1, DOUT))],
        DOUT,
    )(part2, g2s, deg_part, b2r, gm2, bt2, Wp, bpr)

    return out


# speedup vs baseline: 2.8201x; 1.0472x over previous
import functools
import math

import jax
import jax.numpy as jnp
from jax import lax
from jax.experimental import pallas as pl
from jax.experimental.pallas import tpu as pltpu
from jax.experimental.pallas import tpu_sc as plsc

N = 10000
DIN = 128
DH = 128
DOUT = 64
E = 320000
NC = 2
NS = 16
NW = NC * NS
EPT = E // NW
K = 80
NCHUNK = EPT // K
KC = 128
NFULL = EPT // KC
NPAIR = NFULL // 2
KT = EPT - NFULL * KC
RPS = N // NS
RPA = 624
RTAIL = N - NS * RPA
ROWB = 1000
GRID = N // ROWB
BN_SCALE = 1.0 / math.sqrt(1.0 + 1e-5)

_MESH = plsc.VectorSubcoreMesh(core_axis_name="core", subcore_axis_name="subcore")


@functools.partial(
    pl.kernel,
    out_type=jax.ShapeDtypeStruct((NC, N, DH), jnp.float32),
    mesh=_MESH,
    scratch_types=[
        pltpu.VMEM((EPT,), jnp.int32),
        pltpu.VMEM((KC,), jnp.int32),
        pltpu.VMEM((KC,), jnp.int32),
        pltpu.VMEM((KT,), jnp.int32),
        pltpu.VMEM((KC, DH), jnp.float32),
        pltpu.VMEM((KC, DH), jnp.float32),
        pltpu.VMEM((KT, DH), jnp.float32),
        pltpu.VMEM_SHARED((N, DH), jnp.float32),
        pltpu.SemaphoreType.DMA,
        pltpu.SemaphoreType.DMA,
        pltpu.SemaphoreType.DMA,
        pltpu.SemaphoreType.DMA,
    ],
)
def _agg_kernel(g_hbm, src_hbm, dst_hbm, part_hbm, sidx_all, didx0, didx1,
                didxt, rows0, rows1, rowst, acc, sd0, sd1, sg0, sg1):
    c = lax.axis_index("core")
    s = lax.axis_index("subcore")
    wid = c * NS + s
    r0 = s * RPA
    base0 = wid * EPT

    pltpu.sync_copy(g_hbm.at[pl.ds(r0, RPA)], acc.at[pl.ds(r0, RPA)])

    @pl.when(s == NS - 1)
    def _():
        pltpu.sync_copy(g_hbm.at[pl.ds(NS * RPA, RTAIL)],
                        acc.at[pl.ds(NS * RPA, RTAIL)])

    pltpu.sync_copy(src_hbm.at[pl.ds(base0, EPT)], sidx_all)
    plsc.subcore_barrier()

    def _didx(ck, buf, sem):
        return pltpu.make_async_copy(
            dst_hbm.at[pl.ds(base0 + ck * KC, KC)], buf, sem)

    def _gath(ck, buf, sem):
        return pltpu.make_async_copy(
            g_hbm.at[sidx_all.at[pl.ds(ck * KC, KC)]], buf, sem)

    _didx(0, didx0, sd0).start()
    _gath(0, rows0, sg0).start()

    @pl.loop(0, NPAIR)
    def _(t):
        a = 2 * t
        _didx(a, didx0, sd0).wait()
        _gath(a, rows0, sg0).wait()
        _didx(a + 1, didx1, sd1).start()
        _gath(a + 1, rows1, sg1).start()
        pltpu.sync_copy(rows0, acc.at[didx0], add=True)
        _didx(a + 1, didx1, sd1).wait()
        _gath(a + 1, rows1, sg1).wait()

        @pl.when(t < NPAIR - 1)
        def _():
            _didx(a + 2, didx0, sd0).start()
            _gath(a + 2, rows0, sg0).start()

        pltpu.sync_copy(rows1, acc.at[didx1], add=True)

    pltpu.sync_copy(dst_hbm.at[pl.ds(base0 + NFULL * KC, KT)], didxt)
    pltpu.sync_copy(g_hbm.at[sidx_all.at[pl.ds(NFULL * KC, KT)]], rowst)
    pltpu.sync_copy(rowst, acc.at[didxt], add=True)

    plsc.subcore_barrier()
    pltpu.sync_copy(acc.at[pl.ds(r0, RPA)], part_hbm.at[c, pl.ds(r0, RPA)])

    @pl.when(s == NS - 1)
    def _():
        pltpu.sync_copy(acc.at[pl.ds(NS * RPA, RTAIL)],
                        part_hbm.at[c, pl.ds(NS * RPA, RTAIL)])


@functools.partial(
    pl.kernel,
    out_type=jax.ShapeDtypeStruct((NC, N, DH), jnp.float32),
    mesh=_MESH,
    scratch_types=[
        pltpu.VMEM((KC,), jnp.int32),
        pltpu.VMEM((KC,), jnp.int32),
        pltpu.VMEM((KT,), jnp.int32),
        pltpu.VMEM((KC, DH), jnp.float32),
        pltpu.VMEM_SHARED((N, DH), jnp.float32),
        pltpu.SemaphoreType.DMA,
        pltpu.SemaphoreType.DMA,
    ],
)
def _deg_kernel(ones_hbm, dst_hbm, part_hbm, didx0, didx1, didxt, ones,
                acc, sd0, sd1):
    c = lax.axis_index("core")
    s = lax.axis_index("subcore")
    wid = c * NS + s
    r0 = s * RPA
    base0 = wid * EPT

    pltpu.sync_copy(ones_hbm.at[pl.ds(r0, RPA)], acc.at[pl.ds(r0, RPA)])

    @pl.when(s == NS - 1)
    def _():
        pltpu.sync_copy(ones_hbm.at[pl.ds(NS * RPA, RTAIL)],
                        acc.at[pl.ds(NS * RPA, RTAIL)])

    pltpu.sync_copy(ones_hbm.at[pl.ds(0, KC)], ones)
    plsc.subcore_barrier()

    def _didx(ck, buf, sem):
        return pltpu.make_async_copy(
            dst_hbm.at[pl.ds(base0 + ck * KC, KC)], buf, sem)

    _didx(0, didx0, sd0).start()

    @pl.loop(0, NPAIR)
    def _(t):
        a = 2 * t
        _didx(a, didx0, sd0).wait()
        _didx(a + 1, didx1, sd1).start()
        pltpu.sync_copy(ones, acc.at[didx0], add=True)
        _didx(a + 1, didx1, sd1).wait()

        @pl.when(t < NPAIR - 1)
        def _():
            _didx(a + 2, didx0, sd0).start()

        pltpu.sync_copy(ones, acc.at[didx1], add=True)

    pltpu.sync_copy(dst_hbm.at[pl.ds(base0 + NFULL * KC, KT)], didxt)
    pltpu.sync_copy(ones.at[pl.ds(0, KT)], acc.at[didxt], add=True)

    plsc.subcore_barrier()
    pltpu.sync_copy(acc.at[pl.ds(r0, RPA)], part_hbm.at[c, pl.ds(r0, RPA)])

    @pl.when(s == NS - 1)
    def _():
        pltpu.sync_copy(acc.at[pl.ds(NS * RPA, RTAIL)],
                        part_hbm.at[c, pl.ds(NS * RPA, RTAIL)])


def _dis_from_deg(deg_blk):
    d = deg_blk[0][:, 0:1] + deg_blk[1][:, 0:1] - 1.0
    return 1.0 / jnp.sqrt(d)


def _mm(a, b):
    return lax.dot_general(a, b, (((1,), (0,)), ((), ())),
                           precision=lax.Precision.HIGHEST,
                           preferred_element_type=jnp.float32)


def _mm1_body(x_ref, w_ref, deg_ref, g_ref):
    g_ref[...] = _mm(x_ref[...], w_ref[...]) * _dis_from_deg(deg_ref[...])


def _mid_body(part_ref, g_ref, deg_ref, b_ref, gam_ref, bet_ref, w_ref, o_ref):
    dis = _dis_from_deg(deg_ref[...])
    p = part_ref[...]
    t = (p[0] + p[1] - g_ref[...]) * dis + b_ref[...]
    t = t * (BN_SCALE * gam_ref[...]) + bet_ref[...]
    e = jnp.where(t > 0, t, jnp.exp(t) - 1.0)
    o_ref[...] = _mm(e, w_ref[...]) * dis


def _out_body(part_ref, g_ref, deg_ref, b_ref, gam_ref, bet_ref, w_ref, bp_ref, o_ref):
    dis = _dis_from_deg(deg_ref[...])
    p = part_ref[...]
    t = (p[0] + p[1] - g_ref[...]) * dis + b_ref[...]
    t = t * (BN_SCALE * gam_ref[...]) + bet_ref[...]
    e = jnp.where(t > 0, t, jnp.exp(t) - 1.0)
    o_ref[...] = _mm(e, w_ref[...]) + bp_ref[...]


def _rows_spec(d):
    return pl.BlockSpec((ROWB, d), lambda i: (i, 0))


def _full_spec(shape):
    nd = len(shape)
    return pl.BlockSpec(shape, lambda i, _nd=nd: (0,) * _nd)


def _part_spec(d):
    return pl.BlockSpec((NC, ROWB, d), lambda i: (0, i, 0))


_DEG_SPEC = pl.BlockSpec((NC, ROWB, DH), lambda i: (0, i, 0))


def _tc_call(body, in_specs, out_d):
    return pl.pallas_call(
        body,
        grid=(GRID,),
        in_specs=in_specs,
        out_specs=_rows_spec(out_d),
        out_shape=jax.ShapeDtypeStruct((N, out_d), jnp.float32),
    )


def kernel(x, edge_index, W1, b1, g1, be1, W2, b2, g2, be2, Wp, bp):
    src = edge_index[0].astype(jnp.int32)
    dst = edge_index[1].astype(jnp.int32)
    b1r, gm1, bt1 = b1.reshape(1, DH), g1.reshape(1, DH), be1.reshape(1, DH)
    b2r, gm2, bt2 = b2.reshape(1, DH), g2.reshape(1, DH), be2.reshape(1, DH)
    bpr = bp.reshape(1, DOUT)

    ones_n = jnp.ones((N, DH), jnp.float32)
    deg_part = _deg_kernel(ones_n, dst)

    g1s = _tc_call(_mm1_body,
                   [_rows_spec(DIN), _full_spec((DIN, DH)), _DEG_SPEC],
                   DH)(x, W1, deg_part)

    part1 = _agg_kernel(g1s, src, dst)

    g2s = _tc_call(
        _mid_body,
        [_part_spec(DH), _rows_spec(DH), _DEG_SPEC, _full_spec((1, DH)),
         _full_spec((1, DH)), _full_spec((1, DH)), _full_spec((DH, DH))],
        DH,
    )(part1, g1s, deg_part, b1r, gm1, bt1, W2)

    part2 = _agg_kernel(g2s, src, dst)

    out = _tc_call(
        _out_body,
        [_part_spec(DH), _rows_spec(DH), _DEG_SPEC, _full_spec((1, DH)),
         _full_spec((1, DH)), _full_spec((1, DH)), _full_spec((DH, DOUT)),
         _full_spec((1, DOUT))],
        DOUT,
    )(part2, g2s, deg_part, b2r, gm2, bt2, Wp, bpr)

    return out
